# baseline (speedup 1.0000x reference)
import functools

import jax
import jax.numpy as jnp
from jax import lax
from jax.experimental import pallas as pl
from jax.experimental.pallas import tpu as pltpu
from jax.experimental.pallas import tpu_sc as plsc

B = 16384
V = 1000000
F = 64
NC = 2
NS = 16
NW = NC * NS
BPW = B // NW
C = 16
NCH = BPW // C

SLAB = 256
CPB = SLAB // 16
CPB_LOG2 = CPB.bit_length() - 1
H = 450048
TAIL0 = (V // 128) * 128
HI_SLABS = (TAIL0 - H) // SLAB
MAIN_SLABS = HI_SLABS // NW
EXTRA = HI_SLABS - MAIN_SLABS * NW
HI_ROWS = V - H


def _transpose_slab(src, dst):
    lanes = lax.iota(jnp.int32, 16)

    @plsc.parallel_loop(0, (F // 16) * CPB * 16, 1, unroll=16)
    def _(i):
        d = i & 15
        bid = i >> 4
        c0 = (bid & (CPB - 1)) * 16
        f0 = (bid >> CPB_LOG2) * 16
        rot = (lanes + d) & 15
        fvec = lanes + f0
        cvec = rot + c0
        v = plsc.load_gather(src, [fvec, cvec])
        plsc.store_scatter(dst, [cvec, fvec], v)


def _start_read(tT_hbm, sbuf, src_off, sem):
    pltpu.async_copy(
        tT_hbm.at[:, pl.ds(pl.multiple_of(src_off, SLAB), SLAB)], sbuf, sem)


def _wait_read(tT_hbm, sbuf, sem):
    pltpu.make_async_copy(tT_hbm.at[:, pl.ds(0, SLAB)], sbuf, sem).wait()


def _start_write(out_hbm, tbuf, dst_off, sem):
    pltpu.async_copy(
        tbuf, out_hbm.at[pl.ds(pl.multiple_of(dst_off, SLAB), SLAB), :], sem)


def _wait_write(out_hbm, tbuf, sem):
    pltpu.make_async_copy(tbuf, out_hbm.at[pl.ds(0, SLAB), :], sem).wait()


def _tr_body(tT_hbm, tail_hbm, out_hbm,
             sbuf0, sbuf1, tbuf0, tbuf1, sem_r0, sem_r1, sem_w0, sem_w1):
    wid = lax.axis_index("s") * NC + lax.axis_index("c")
    d0 = wid * (MAIN_SLABS * SLAB)
    sb = (sbuf0, sbuf1)
    tb = (tbuf0, tbuf1)
    sr = (sem_r0, sem_r1)
    sw = (sem_w0, sem_w1)

    _start_read(tT_hbm, sbuf0, H + d0, sem_r0)
    _start_read(tT_hbm, sbuf1, H + d0 + SLAB, sem_r1)

    def step(k, carry):
        for b in (0, 1):
            off = d0 + (k * 2 + b) * SLAB
            _wait_read(tT_hbm, sb[b], sr[b])

            @pl.when(k > 0)
            def _():
                _wait_write(out_hbm, tb[b], sw[b])

            _transpose_slab(sb[b], tb[b])
            _start_write(out_hbm, tb[b], off, sw[b])

            if b == 0:
                _start_read(tT_hbm, sb[b], H + off + 2 * SLAB, sr[b])
            else:
                @pl.when(k < MAIN_SLABS // 2 - 1)
                def _():
                    _start_read(tT_hbm, sb[b], H + off + 2 * SLAB, sr[b])

        return carry

    lax.fori_loop(0, MAIN_SLABS // 2, step, 0)
    last = d0 + (MAIN_SLABS - 1) * SLAB
    _wait_read(tT_hbm, sbuf0, sem_r0)
    _wait_write(out_hbm, tbuf0, sem_w0)
    _transpose_slab(sbuf0, tbuf0)
    _start_write(out_hbm, tbuf0, last, sem_w0)
    _wait_write(out_hbm, tbuf0, sem_w0)
    _wait_write(out_hbm, tbuf1, sem_w1)

    @pl.when(wid < EXTRA)
    def _():
        off = (MAIN_SLABS * NW + wid) * SLAB
        pltpu.sync_copy(
            tT_hbm.at[:, pl.ds(pl.multiple_of(H + off, SLAB), SLAB)], sbuf0)
        _transpose_slab(sbuf0, tbuf0)
        pltpu.sync_copy(
            tbuf0, out_hbm.at[pl.ds(pl.multiple_of(off, SLAB), SLAB), :])

    @pl.when(wid == NW - 1)
    def _():
        pltpu.sync_copy(tail_hbm, tbuf0.at[pl.ds(0, V - TAIL0), :])
        pltpu.sync_copy(tbuf0.at[pl.ds(0, V - TAIL0), :],
                        out_hbm.at[pl.ds(TAIL0 - H, V - TAIL0), :])


@functools.partial(
    pl.kernel,
    out_type=jax.ShapeDtypeStruct((HI_ROWS, F), jnp.float32),
    mesh=plsc.VectorSubcoreMesh(core_axis_name="c", subcore_axis_name="s"),
    compiler_params=pltpu.CompilerParams(
        needs_layout_passes=False, use_tc_tiling_on_sc=True),
    scratch_types=[
        pltpu.VMEM((F, SLAB), jnp.float32),
        pltpu.VMEM((F, SLAB), jnp.float32),
        pltpu.VMEM((SLAB, F), jnp.float32),
        pltpu.VMEM((SLAB, F), jnp.float32),
        pltpu.SemaphoreType.DMA,
        pltpu.SemaphoreType.DMA,
        pltpu.SemaphoreType.DMA,
        pltpu.SemaphoreType.DMA,
    ],
)
def _transpose_tbl(tT_hbm, tail_hbm, out_hbm,
                   sbuf0, sbuf1, tbuf0, tbuf1,
                   sem_r0, sem_r1, sem_w0, sem_w1):
    _tr_body(tT_hbm, tail_hbm, out_hbm,
             sbuf0, sbuf1, tbuf0, tbuf1, sem_r0, sem_r1, sem_w0, sem_w1)


def _start_fetches(lo_hbm, hi_hbm, idxs, bufs, b, chunk, sem):
    for t in range(3):
        vidx = idxs[t][pl.ds(chunk * C, C)]
        vlo = jnp.minimum(vidx, H - 1)
        vhi = jnp.maximum(vidx - H, 0)
        for i in range(C):
            pltpu.async_copy(lo_hbm.at[vlo[i]], bufs.at[b, 0, t, i], sem)
            pltpu.async_copy(hi_hbm.at[vhi[i]], bufs.at[b, 1, t, i], sem)


def _drain_fetches(lo_hbm, bufs, b, sem):
    for h in range(2):
        for t in range(3):
            for i in range(C):
                pltpu.make_async_copy(
                    lo_hbm.at[0], bufs.at[b, h, t, i], sem).wait()


def _compute_chunk(idxs, bufs, b, chunk, outbuf):
    lane = lax.iota(jnp.int32, 16)
    tot = jnp.zeros((16,), jnp.float32)
    is_lo = [(idxs[t][pl.ds(chunk * C, C)] < H).astype(jnp.int32)
             for t in range(3)]
    for i in range(C):
        rows = []
        for t in range(3):
            sel = is_lo[t][i] > 0
            rows.append([
                jnp.where(sel,
                          bufs[b, 0, t, i, pl.ds(j * 16, 16)],
                          bufs[b, 1, t, i, pl.ds(j * 16, 16)])
                for j in range(F // 16)])
        parts = [rows[0][j] * rows[1][j] * rows[2][j] for j in range(F // 16)]
        s = (parts[0] + parts[1]) + (parts[2] + parts[3])
        tot = jnp.where(lane == i, jnp.sum(s), tot)
    outbuf[pl.ds(chunk * C, C)] = tot


def _sc_body(user_hbm, item_hbm, ctx_hbm, lo_hbm, hi_hbm, out_hbm,
             idx_u, idx_i, idx_c, bufs, outbuf, sem_idx, sem0, sem1):
    idxs = (idx_u, idx_i, idx_c)
    wid = lax.axis_index("s") * NC + lax.axis_index("c")
    base = wid * BPW

    cps = [
        pltpu.async_copy(user_hbm.at[pl.ds(base, BPW)], idx_u, sem_idx),
        pltpu.async_copy(item_hbm.at[pl.ds(base, BPW)], idx_i, sem_idx),
        pltpu.async_copy(ctx_hbm.at[pl.ds(base, BPW)], idx_c, sem_idx),
    ]
    for cp in cps:
        cp.wait()

    _start_fetches(lo_hbm, hi_hbm, idxs, bufs, 0, 0, sem0)

    def pipe(k, carry):
        g = k * 2
        _start_fetches(lo_hbm, hi_hbm, idxs, bufs, 1, g + 1, sem1)
        _drain_fetches(lo_hbm, bufs, 0, sem0)
        _compute_chunk(idxs, bufs, 0, g, outbuf)

        @pl.when(g + 2 < NCH)
        def _():
            _start_fetches(lo_hbm, hi_hbm, idxs, bufs, 0, g + 2, sem0)

        _drain_fetches(lo_hbm, bufs, 1, sem1)
        _compute_chunk(idxs, bufs, 1, g + 1, outbuf)
        return carry

    lax.fori_loop(0, NCH // 2, pipe, 0)

    pltpu.sync_copy(outbuf, out_hbm.at[pl.ds(base, BPW)])


@functools.partial(
    pl.kernel,
    out_type=jax.ShapeDtypeStruct((B,), jnp.float32),
    mesh=plsc.VectorSubcoreMesh(core_axis_name="c", subcore_axis_name="s"),
    compiler_params=pltpu.CompilerParams(
        needs_layout_passes=False, use_tc_tiling_on_sc=True),
    scratch_types=[
        pltpu.VMEM((BPW,), jnp.int32),
        pltpu.VMEM((BPW,), jnp.int32),
        pltpu.VMEM((BPW,), jnp.int32),
        pltpu.VMEM((2, 2, 3, C, F), jnp.float32),
        pltpu.VMEM((BPW,), jnp.float32),
        pltpu.SemaphoreType.DMA,
        pltpu.SemaphoreType.DMA,
        pltpu.SemaphoreType.DMA,
    ],
)
def _pointmf_sc(user_hbm, item_hbm, ctx_hbm, lo_hbm, hi_hbm, out_hbm,
                idx_u, idx_i, idx_c, bufs, outbuf, sem_idx, sem0, sem1):
    _sc_body(user_hbm, item_hbm, ctx_hbm, lo_hbm, hi_hbm, out_hbm,
             idx_u, idx_i, idx_c, bufs, outbuf, sem_idx, sem0, sem1)


def kernel(user, item, context, table):
    lo = lax.slice(table, (0, 0), (H, F))
    tail = lax.slice(table, (TAIL0, 0), (V, F))
    hi = _transpose_tbl(table.T, tail)
    return _pointmf_sc(user.astype(jnp.int32), item.astype(jnp.int32),
                       context.astype(jnp.int32), lo, hi)


# speedup vs baseline: 4.2057x; 4.2057x over previous
import functools

import jax
import jax.numpy as jnp
from jax import lax
from jax.experimental import pallas as pl
from jax.experimental.pallas import tpu as pltpu
from jax.experimental.pallas import tpu_sc as plsc

B = 16384
V = 1000000
F = 64
NC = 2
NS = 16
NW = NC * NS
BPW = B // NW
C = 16
NCH = BPW // C

SLAB = 256
CPB = SLAB // 16
CPB_LOG2 = CPB.bit_length() - 1
MAIN_SLABS = 122
NS_FULL = (V // 128) * 128 // SLAB
EXTRA0 = NW * MAIN_SLABS
TAIL0 = NS_FULL * SLAB


def _transpose_slab(src, dst):
    lanes = lax.iota(jnp.int32, 16)

    @plsc.parallel_loop(0, (F // 32) * CPB * 16, 1, unroll=16)
    def _(i):
        d = i & 15
        bid = i >> 4
        c0 = (bid & (CPB - 1)) * 16
        h0 = (bid >> CPB_LOG2) * 16
        rot = (lanes + d) & 15
        wvec = lanes + h0
        fvec_e = wvec * 2
        cvec = rot + c0
        ve = plsc.load_gather(src, [fvec_e, cvec])
        vo = plsc.load_gather(src, [fvec_e + 1, cvec])
        packed = plsc.pack(ve, vo, format=plsc.PackFormat.INTERLEAVED)
        plsc.store_scatter(dst, [cvec, wvec], plsc.bitcast(packed, jnp.int32))


def _start_read(tT_hbm, sbuf, off, sem):
    pltpu.async_copy(
        tT_hbm.at[:, pl.ds(pl.multiple_of(off, SLAB), SLAB)], sbuf, sem)


def _wait_read(tT_hbm, sbuf, sem):
    pltpu.make_async_copy(tT_hbm.at[:, pl.ds(0, SLAB)], sbuf, sem).wait()


def _start_write(out_hbm, tbuf, off, sem):
    pltpu.async_copy(
        tbuf, out_hbm.at[pl.ds(pl.multiple_of(off, SLAB), SLAB), :], sem)


def _wait_write(out_hbm, tbuf, sem):
    pltpu.make_async_copy(tbuf, out_hbm.at[pl.ds(0, SLAB), :], sem).wait()


def _tr_body(tT_hbm, tail_hbm, out_hbm,
             sbuf0, sbuf1, tbuf0, tbuf1, tailbuf,
             sem_r0, sem_r1, sem_w0, sem_w1):
    wid = lax.axis_index("s") * NC + lax.axis_index("c")
    g0 = wid * (MAIN_SLABS * SLAB)
    sb = (sbuf0, sbuf1)
    tb = (tbuf0, tbuf1)
    sr = (sem_r0, sem_r1)
    sw = (sem_w0, sem_w1)

    _start_read(tT_hbm, sbuf0, g0, sem_r0)
    _start_read(tT_hbm, sbuf1, g0 + SLAB, sem_r1)

    def step(k, carry):
        for b in (0, 1):
            off = g0 + (k * 2 + b) * SLAB
            _wait_read(tT_hbm, sb[b], sr[b])

            @pl.when(k > 0)
            def _():
                _wait_write(out_hbm, tb[b], sw[b])

            _transpose_slab(sb[b], tb[b])
            _start_write(out_hbm, tb[b], off, sw[b])

            @pl.when(k < MAIN_SLABS // 2 - 1)
            def _():
                _start_read(tT_hbm, sb[b], off + 2 * SLAB, sr[b])

        return carry

    lax.fori_loop(0, MAIN_SLABS // 2, step, 0)
    _wait_write(out_hbm, tbuf0, sem_w0)
    _wait_write(out_hbm, tbuf1, sem_w1)

    @pl.when(wid < NS_FULL - EXTRA0)
    def _():
        off = (EXTRA0 + wid) * SLAB
        pltpu.sync_copy(
            tT_hbm.at[:, pl.ds(pl.multiple_of(off, SLAB), SLAB)], sbuf0)
        _transpose_slab(sbuf0, tbuf0)
        pltpu.sync_copy(
            tbuf0, out_hbm.at[pl.ds(pl.multiple_of(off, SLAB), SLAB), :])

    @pl.when(wid == NW - 1)
    def _():
        lanes = lax.iota(jnp.int32, 16)
        pltpu.sync_copy(tail_hbm, tailbuf)

        @plsc.parallel_loop(0, (V - TAIL0) * (F // 32), 1, unroll=8)
        def _(i):
            r = i >> 1
            wb = i & 1
            zr = jnp.zeros((16,), jnp.int32) + r
            fvec_e = 2 * lanes + wb * 32
            ve = plsc.load_gather(tailbuf, [zr, fvec_e])
            vo = plsc.load_gather(tailbuf, [zr, fvec_e + 1])
            packed = plsc.pack(ve, vo, format=plsc.PackFormat.INTERLEAVED)
            plsc.store_scatter(tbuf0, [zr, lanes + wb * 16],
                               plsc.bitcast(packed, jnp.int32))

        pltpu.sync_copy(tbuf0.at[pl.ds(0, V - TAIL0), :],
                        out_hbm.at[pl.ds(TAIL0, V - TAIL0), :])


@functools.partial(
    pl.kernel,
    out_type=jax.ShapeDtypeStruct((V, F // 2), jnp.int32),
    mesh=plsc.VectorSubcoreMesh(core_axis_name="c", subcore_axis_name="s"),
    compiler_params=pltpu.CompilerParams(
        needs_layout_passes=False, use_tc_tiling_on_sc=True),
    scratch_types=[
        pltpu.VMEM((F, SLAB), jnp.float32),
        pltpu.VMEM((F, SLAB), jnp.float32),
        pltpu.VMEM((SLAB, F // 2), jnp.int32),
        pltpu.VMEM((SLAB, F // 2), jnp.int32),
        pltpu.VMEM((V - TAIL0, F), jnp.float32),
        pltpu.SemaphoreType.DMA,
        pltpu.SemaphoreType.DMA,
        pltpu.SemaphoreType.DMA,
        pltpu.SemaphoreType.DMA,
    ],
)
def _transpose_tbl(tT_hbm, tail_hbm, out_hbm,
                   sbuf0, sbuf1, tbuf0, tbuf1, tailbuf,
                   sem_r0, sem_r1, sem_w0, sem_w1):
    _tr_body(tT_hbm, tail_hbm, out_hbm,
             sbuf0, sbuf1, tbuf0, tbuf1, tailbuf,
             sem_r0, sem_r1, sem_w0, sem_w1)


def _start_fetches(table_hbm, idxs, bufs, b, chunk, sem):
    for t in range(3):
        vidx = idxs[t][pl.ds(chunk * C, C)]
        for i in range(C):
            pltpu.async_copy(table_hbm.at[vidx[i]], bufs.at[b, t, i], sem)


def _drain_fetches(table_hbm, bufs, b, sem):
    for t in range(3):
        for i in range(C):
            pltpu.make_async_copy(
                table_hbm.at[0], bufs.at[b, t, i], sem).wait()


def _unpack_row(bufs, b, t, i):
    pieces = []
    for w in range(F // 32):
        word = bufs[b, t, i, pl.ds(w * 16, 16)]
        bf = plsc.bitcast(word, jnp.bfloat16)
        e, o = plsc.unpack(bf, format=plsc.PackFormat.INTERLEAVED)
        pieces += [e, o]
    return pieces


def _compute_chunk(bufs, b, chunk, outbuf):
    lane = lax.iota(jnp.int32, 16)
    tot = jnp.zeros((16,), jnp.float32)
    for i in range(C):
        rows = [_unpack_row(bufs, b, t, i) for t in range(3)]
        parts = [rows[0][j] * rows[1][j] * rows[2][j] for j in range(F // 16)]
        s = (parts[0] + parts[1]) + (parts[2] + parts[3])
        tot = jnp.where(lane == i, jnp.sum(s), tot)
    outbuf[pl.ds(chunk * C, C)] = tot


def _sc_body(user_hbm, item_hbm, ctx_hbm, table_hbm, out_hbm,
             idx_u, idx_i, idx_c, bufs, outbuf, sem_idx, sem0, sem1):
    idxs = (idx_u, idx_i, idx_c)
    wid = lax.axis_index("s") * NC + lax.axis_index("c")
    base = wid * BPW

    cps = [
        pltpu.async_copy(user_hbm.at[pl.ds(base, BPW)], idx_u, sem_idx),
        pltpu.async_copy(item_hbm.at[pl.ds(base, BPW)], idx_i, sem_idx),
        pltpu.async_copy(ctx_hbm.at[pl.ds(base, BPW)], idx_c, sem_idx),
    ]
    for cp in cps:
        cp.wait()

    _start_fetches(table_hbm, idxs, bufs, 0, 0, sem0)

    def pipe(k, carry):
        g = k * 2
        _start_fetches(table_hbm, idxs, bufs, 1, g + 1, sem1)
        _drain_fetches(table_hbm, bufs, 0, sem0)
        _compute_chunk(bufs, 0, g, outbuf)

        @pl.when(g + 2 < NCH)
        def _():
            _start_fetches(table_hbm, idxs, bufs, 0, g + 2, sem0)

        _drain_fetches(table_hbm, bufs, 1, sem1)
        _compute_chunk(bufs, 1, g + 1, outbuf)
        return carry

    lax.fori_loop(0, NCH // 2, pipe, 0)

    pltpu.sync_copy(outbuf, out_hbm.at[pl.ds(base, BPW)])


@functools.partial(
    pl.kernel,
    out_type=jax.ShapeDtypeStruct((B,), jnp.float32),
    mesh=plsc.VectorSubcoreMesh(core_axis_name="c", subcore_axis_name="s"),
    compiler_params=pltpu.CompilerParams(
        needs_layout_passes=False, use_tc_tiling_on_sc=True),
    scratch_types=[
        pltpu.VMEM((BPW,), jnp.int32),
        pltpu.VMEM((BPW,), jnp.int32),
        pltpu.VMEM((BPW,), jnp.int32),
        pltpu.VMEM((2, 3, C, F // 2), jnp.int32),
        pltpu.VMEM((BPW,), jnp.float32),
        pltpu.SemaphoreType.DMA,
        pltpu.SemaphoreType.DMA,
        pltpu.SemaphoreType.DMA,
    ],
)
def _pointmf_sc(user_hbm, item_hbm, ctx_hbm, table_hbm, out_hbm,
                idx_u, idx_i, idx_c, bufs, outbuf, sem_idx, sem0, sem1):
    _sc_body(user_hbm, item_hbm, ctx_hbm, table_hbm, out_hbm,
             idx_u, idx_i, idx_c, bufs, outbuf, sem_idx, sem0, sem1)


def kernel(user, item, context, table):
    tail = lax.slice(table, (TAIL0, 0), (V, F))
    table_rm = _transpose_tbl(table.T, tail)
    return _pointmf_sc(user.astype(jnp.int32), item.astype(jnp.int32),
                       context.astype(jnp.int32), table_rm)


# speedup vs baseline: 4.2916x; 1.0204x over previous
import functools

import jax
import jax.numpy as jnp
from jax import lax
from jax.experimental import pallas as pl
from jax.experimental.pallas import tpu as pltpu
from jax.experimental.pallas import tpu_sc as plsc

B = 16384
V = 1000000
F = 64
NC = 2
NS = 16
NW = NC * NS
BPW = B // NW
C = 16
NCH = BPW // C

SLAB = 256
CPB = SLAB // 16
CPB_LOG2 = CPB.bit_length() - 1
MAIN_SLABS = 122
NS_FULL = (V // 128) * 128 // SLAB
EXTRA0 = NW * MAIN_SLABS
TAIL0 = NS_FULL * SLAB


def _transpose_slab(src, dst):
    lanes = lax.iota(jnp.int32, 16)

    @plsc.parallel_loop(0, (F // 16) * CPB * 16, 1, unroll=16)
    def _(i):
        d = i & 15
        bid = i >> 4
        c0 = (bid & (CPB - 1)) * 16
        f0 = (bid >> CPB_LOG2) * 16
        rot = (lanes + d) & 15
        fvec = lanes + f0
        cvec = rot + c0
        v = plsc.load_gather(src, [fvec, cvec])
        plsc.store_scatter(dst, [cvec, fvec], v)


def _start_read(tT_hbm, sbuf, off, sem):
    pltpu.async_copy(
        tT_hbm.at[:, pl.ds(pl.multiple_of(off, SLAB), SLAB)], sbuf, sem)


def _wait_read(tT_hbm, sbuf, sem):
    pltpu.make_async_copy(tT_hbm.at[:, pl.ds(0, SLAB)], sbuf, sem).wait()


def _start_write(out_hbm, tbuf, off, sem):
    pltpu.async_copy(
        tbuf, out_hbm.at[pl.ds(pl.multiple_of(off, SLAB), SLAB), :], sem)


def _wait_write(out_hbm, tbuf, sem):
    pltpu.make_async_copy(tbuf, out_hbm.at[pl.ds(0, SLAB), :], sem).wait()


def _tr_body(tT_hbm, tail_hbm, out_hbm,
             sbuf0, sbuf1, tbuf0, tbuf1, sem_r0, sem_r1, sem_w0, sem_w1):
    wid = lax.axis_index("s") * NC + lax.axis_index("c")
    g0 = wid * (MAIN_SLABS * SLAB)
    sb = (sbuf0, sbuf1)
    tb = (tbuf0, tbuf1)
    sr = (sem_r0, sem_r1)
    sw = (sem_w0, sem_w1)

    _start_read(tT_hbm, sbuf0, g0, sem_r0)
    _start_read(tT_hbm, sbuf1, g0 + SLAB, sem_r1)

    def step(k, carry):
        for b in (0, 1):
            off = g0 + (k * 2 + b) * SLAB
            _wait_read(tT_hbm, sb[b], sr[b])

            @pl.when(k > 0)
            def _():
                _wait_write(out_hbm, tb[b], sw[b])

            _transpose_slab(sb[b], tb[b])
            _start_write(out_hbm, tb[b], off, sw[b])

            @pl.when(k < MAIN_SLABS // 2 - 1)
            def _():
                _start_read(tT_hbm, sb[b], off + 2 * SLAB, sr[b])

        return carry

    lax.fori_loop(0, MAIN_SLABS // 2, step, 0)
    _wait_write(out_hbm, tbuf0, sem_w0)
    _wait_write(out_hbm, tbuf1, sem_w1)

    @pl.when(wid < NS_FULL - EXTRA0)
    def _():
        off = (EXTRA0 + wid) * SLAB
        pltpu.sync_copy(
            tT_hbm.at[:, pl.ds(pl.multiple_of(off, SLAB), SLAB)], sbuf0)
        _transpose_slab(sbuf0, tbuf0)
        pltpu.sync_copy(
            tbuf0, out_hbm.at[pl.ds(pl.multiple_of(off, SLAB), SLAB), :])

    @pl.when(wid == NW - 1)
    def _():
        pltpu.sync_copy(tail_hbm, tbuf0.at[pl.ds(0, V - TAIL0), :])
        pltpu.sync_copy(tbuf0.at[pl.ds(0, V - TAIL0), :],
                        out_hbm.at[pl.ds(TAIL0, V - TAIL0), :])


@functools.partial(
    pl.kernel,
    out_type=jax.ShapeDtypeStruct((V, F), jnp.float32),
    mesh=plsc.VectorSubcoreMesh(core_axis_name="c", subcore_axis_name="s"),
    compiler_params=pltpu.CompilerParams(
        needs_layout_passes=False, use_tc_tiling_on_sc=True),
    scratch_types=[
        pltpu.VMEM((F, SLAB), jnp.float32),
        pltpu.VMEM((F, SLAB), jnp.float32),
        pltpu.VMEM((SLAB, F), jnp.float32),
        pltpu.VMEM((SLAB, F), jnp.float32),
        pltpu.SemaphoreType.DMA,
        pltpu.SemaphoreType.DMA,
        pltpu.SemaphoreType.DMA,
        pltpu.SemaphoreType.DMA,
    ],
)
def _transpose_tbl(tT_hbm, tail_hbm, out_hbm,
                   sbuf0, sbuf1, tbuf0, tbuf1,
                   sem_r0, sem_r1, sem_w0, sem_w1):
    _tr_body(tT_hbm, tail_hbm, out_hbm,
             sbuf0, sbuf1, tbuf0, tbuf1, sem_r0, sem_r1, sem_w0, sem_w1)


def _start_fetches(table_hbm, idxs, bufs, b, chunk, sem):
    for t in range(3):
        vidx = idxs[t][pl.ds(chunk * C, C)]
        for i in range(C):
            pltpu.async_copy(table_hbm.at[vidx[i]], bufs.at[b, t, i], sem)


def _drain_fetches(table_hbm, bufs, b, sem):
    for t in range(3):
        for i in range(C):
            pltpu.make_async_copy(
                table_hbm.at[0], bufs.at[b, t, i], sem).wait()


def _compute_chunk(bufs, b, chunk, outbuf):
    lane = lax.iota(jnp.int32, 16)
    tot = jnp.zeros((16,), jnp.float32)
    for i in range(C):
        rows = [[bufs[b, t, i, pl.ds(j * 16, 16)] for j in range(F // 16)]
                for t in range(3)]
        parts = [rows[0][j] * rows[1][j] * rows[2][j] for j in range(F // 16)]
        s = (parts[0] + parts[1]) + (parts[2] + parts[3])
        tot = jnp.where(lane == i, jnp.sum(s), tot)
    outbuf[pl.ds(chunk * C, C)] = tot


def _sc_body(user_hbm, item_hbm, ctx_hbm, table_hbm, out_hbm,
             idx_u, idx_i, idx_c, bufs, outbuf, sem_idx, sem0, sem1):
    idxs = (idx_u, idx_i, idx_c)
    wid = lax.axis_index("s") * NC + lax.axis_index("c")
    base = wid * BPW

    cps = [
        pltpu.async_copy(user_hbm.at[pl.ds(base, BPW)], idx_u, sem_idx),
        pltpu.async_copy(item_hbm.at[pl.ds(base, BPW)], idx_i, sem_idx),
        pltpu.async_copy(ctx_hbm.at[pl.ds(base, BPW)], idx_c, sem_idx),
    ]
    for cp in cps:
        cp.wait()

    _start_fetches(table_hbm, idxs, bufs, 0, 0, sem0)

    def pipe(k, carry):
        g = k * 2
        _start_fetches(table_hbm, idxs, bufs, 1, g + 1, sem1)
        _drain_fetches(table_hbm, bufs, 0, sem0)
        _compute_chunk(bufs, 0, g, outbuf)

        @pl.when(g + 2 < NCH)
        def _():
            _start_fetches(table_hbm, idxs, bufs, 0, g + 2, sem0)

        _drain_fetches(table_hbm, bufs, 1, sem1)
        _compute_chunk(bufs, 1, g + 1, outbuf)
        return carry

    lax.fori_loop(0, NCH // 2, pipe, 0)

    pltpu.sync_copy(outbuf, out_hbm.at[pl.ds(base, BPW)])


@functools.partial(
    pl.kernel,
    out_type=jax.ShapeDtypeStruct((B,), jnp.float32),
    mesh=plsc.VectorSubcoreMesh(core_axis_name="c", subcore_axis_name="s"),
    compiler_params=pltpu.CompilerParams(
        needs_layout_passes=False, use_tc_tiling_on_sc=True),
    scratch_types=[
        pltpu.VMEM((BPW,), jnp.int32),
        pltpu.VMEM((BPW,), jnp.int32),
        pltpu.VMEM((BPW,), jnp.int32),
        pltpu.VMEM((2, 3, C, F), jnp.float32),
        pltpu.VMEM((BPW,), jnp.float32),
        pltpu.SemaphoreType.DMA,
        pltpu.SemaphoreType.DMA,
        pltpu.SemaphoreType.DMA,
    ],
)
def _pointmf_sc(user_hbm, item_hbm, ctx_hbm, table_hbm, out_hbm,
                idx_u, idx_i, idx_c, bufs, outbuf, sem_idx, sem0, sem1):
    _sc_body(user_hbm, item_hbm, ctx_hbm, table_hbm, out_hbm,
             idx_u, idx_i, idx_c, bufs, outbuf, sem_idx, sem0, sem1)


def kernel(user, item, context, table):
    tail = lax.slice(table, (TAIL0, 0), (V, F))
    table_rm = _transpose_tbl(table.T, tail)
    return _pointmf_sc(user.astype(jnp.int32), item.astype(jnp.int32),
                       context.astype(jnp.int32), table_rm)
